# baseline (speedup 1.0000x reference)
import functools

import jax
import jax.numpy as jnp
from jax import lax
from jax.experimental import pallas as pl
from jax.experimental.pallas import tpu as pltpu
from jax.experimental.pallas import tpu_sc as plsc

VOCAB = 100000
EMBED_DIM = 32
BATCH = 1024
CTX = 4

NUM_SC = 2
NUM_SUBCORES = 16
NUM_WORKERS = NUM_SC * NUM_SUBCORES
TOTAL_IDX = BATCH * CTX
IDX_PER_W = TOTAL_IDX // NUM_WORKERS

B_BLK = 32
NSTEP = BATCH // B_BLK
NBUF = 3


def _sc_gather(idx_hbm, table_hbm, out_hbm, idx_v, rows_v, sem):
    wid = lax.axis_index("s") * NUM_SC + lax.axis_index("c")
    base = wid * IDX_PER_W
    pltpu.sync_copy(idx_hbm.at[pl.ds(base, IDX_PER_W)], idx_v)
    pltpu.async_copy(table_hbm.at[idx_v], rows_v, sem).wait()
    pltpu.sync_copy(rows_v, out_hbm.at[pl.ds(base, IDX_PER_W)])


@functools.cache
def _gather_call():
    return pl.kernel(
        _sc_gather,
        out_type=jax.ShapeDtypeStruct((TOTAL_IDX, EMBED_DIM), jnp.float32),
        mesh=plsc.VectorSubcoreMesh(core_axis_name="c", subcore_axis_name="s"),
        scratch_types=[
            pltpu.VMEM((IDX_PER_W,), jnp.int32),
            pltpu.VMEM((IDX_PER_W, EMBED_DIM), jnp.float32),
            pltpu.SemaphoreType.DMA,
        ],
        compiler_params=pltpu.CompilerParams(use_tc_tiling_on_sc=False),
    )


W_CHUNK = 8192


def _tr_body(w_ref, wt_ref):
    wt_ref[...] = w_ref[...].T


def _transpose_w(W):
    grid = (VOCAB + W_CHUNK - 1) // W_CHUNK
    return pl.pallas_call(
        _tr_body,
        grid=(grid,),
        in_specs=[pl.BlockSpec((W_CHUNK, EMBED_DIM), lambda i: (i, 0))],
        out_specs=pl.BlockSpec((EMBED_DIM, W_CHUNK), lambda i: (0, i)),
        out_shape=jax.ShapeDtypeStruct((EMBED_DIM, VOCAB), jnp.float32),
    )(W)


def _tc_body(emb_ref, w_hbm, b_ref, out_hbm, wv, buf, wsem, sems):
    i = pl.program_id(0)

    @pl.when(i == 0)
    def _load_w():
        cp = pltpu.make_async_copy(w_hbm, wv, wsem)
        cp.start()
        cp.wait()

    e = emb_ref[...]
    acc = jnp.zeros((B_BLK, EMBED_DIM), jnp.float32)
    for c in range(CTX):
        chunk = e[:, c * EMBED_DIM:(c + 1) * EMBED_DIM]
        n2 = jnp.sum(chunk * chunk, axis=1, keepdims=True)
        scale = jnp.minimum(1.0, 1.0 / (jnp.sqrt(n2) + 1e-7))
        acc = acc + chunk * scale
    h = acc * (1.0 / CTX)

    slot = lax.rem(i, NBUF)

    def _out_copy(s, step):
        return pltpu.make_async_copy(
            buf.at[s], out_hbm.at[pl.ds(step * B_BLK, B_BLK)], sems.at[s])

    @pl.when(i >= NBUF)
    def _reclaim():
        _out_copy(slot, i - NBUF).wait()

    buf[slot] = lax.dot_general(
        h, wv[...],
        dimension_numbers=(((1,), (0,)), ((), ())),
        preferred_element_type=jnp.float32,
    ) + b_ref[...]
    for k in range(NBUF):
        @pl.when(slot == k)
        def _start(k=k):
            _out_copy(k, i).start(priority=k % 2)

    @pl.when(i == NSTEP - 1)
    def _drain():
        for k in range(NBUF):
            step = NSTEP - NBUF + ((k - NSTEP) % NBUF)
            _out_copy(k, step).wait()


def kernel(x, table, W, b):
    idx = x.reshape(TOTAL_IDX)
    emb = _gather_call()(idx, table)
    emb2d = emb.reshape(BATCH, CTX * EMBED_DIM)

    logits = pl.pallas_call(
        _tc_body,
        grid=(NSTEP,),
        in_specs=[
            pl.BlockSpec((B_BLK, CTX * EMBED_DIM), lambda i: (i, 0)),
            pl.BlockSpec(memory_space=pl.ANY),
            pl.BlockSpec((1, VOCAB), lambda i: (0, 0)),
        ],
        out_specs=pl.BlockSpec(memory_space=pl.ANY),
        out_shape=jax.ShapeDtypeStruct((BATCH, VOCAB), jnp.float32),
        scratch_shapes=[
            pltpu.VMEM((EMBED_DIM, VOCAB), jnp.float32),
            pltpu.VMEM((NBUF, B_BLK, VOCAB), jnp.float32),
            pltpu.SemaphoreType.DMA,
            pltpu.SemaphoreType.DMA((NBUF,)),
        ],
        compiler_params=pltpu.CompilerParams(
            dimension_semantics=("arbitrary",)),
    )(emb2d, _transpose_w(W), b.reshape(1, VOCAB))
    return logits


# speedup vs baseline: 1.0488x; 1.0488x over previous
import functools

import jax
import jax.numpy as jnp
from jax import lax
from jax.experimental import pallas as pl
from jax.experimental.pallas import tpu as pltpu
from jax.experimental.pallas import tpu_sc as plsc

VOCAB = 100000
EMBED_DIM = 32
BATCH = 1024
CTX = 4

NUM_SC = 2
NUM_SUBCORES = 16
NUM_WORKERS = NUM_SC * NUM_SUBCORES
TOTAL_IDX = BATCH * CTX
IDX_PER_W = TOTAL_IDX // NUM_WORKERS

B_BLK = 32
NSTEP = BATCH // B_BLK
NBUF = 3
V_MAIN = (VOCAB // 128) * 128
V_REM = VOCAB - V_MAIN


def _sc_gather(idx_hbm, table_hbm, out_hbm, idx_v, rows_v, sem):
    wid = lax.axis_index("s") * NUM_SC + lax.axis_index("c")
    base = wid * IDX_PER_W
    pltpu.sync_copy(idx_hbm.at[pl.ds(base, IDX_PER_W)], idx_v)
    pltpu.async_copy(table_hbm.at[idx_v], rows_v, sem).wait()
    pltpu.sync_copy(rows_v, out_hbm.at[pl.ds(base, IDX_PER_W)])


@functools.cache
def _gather_call():
    return pl.kernel(
        _sc_gather,
        out_type=jax.ShapeDtypeStruct((TOTAL_IDX, EMBED_DIM), jnp.float32),
        mesh=plsc.VectorSubcoreMesh(core_axis_name="c", subcore_axis_name="s"),
        scratch_types=[
            pltpu.VMEM((IDX_PER_W,), jnp.int32),
            pltpu.VMEM((IDX_PER_W, EMBED_DIM), jnp.float32),
            pltpu.SemaphoreType.DMA,
        ],
        compiler_params=pltpu.CompilerParams(use_tc_tiling_on_sc=False),
    )


W_CHUNK = 8192


def _tr_body(w_ref, wt_ref):
    wt_ref[...] = w_ref[...].T


def _transpose_w(W):
    grid = (VOCAB + W_CHUNK - 1) // W_CHUNK
    return pl.pallas_call(
        _tr_body,
        grid=(grid,),
        in_specs=[pl.BlockSpec((W_CHUNK, EMBED_DIM), lambda i: (i, 0))],
        out_specs=pl.BlockSpec((EMBED_DIM, W_CHUNK), lambda i: (0, i)),
        out_shape=jax.ShapeDtypeStruct((EMBED_DIM, VOCAB), jnp.float32),
    )(W)


def _tc_body(emb_ref, w_hbm, b_ref, out_hbm, wv, buf, rem, wsem, sems):
    i = pl.program_id(0)

    @pl.when(i == 0)
    def _load_w():
        cp = pltpu.make_async_copy(w_hbm, wv, wsem)
        cp.start()
        cp.wait()

    e = emb_ref[...]
    acc = jnp.zeros((B_BLK, EMBED_DIM), jnp.float32)
    for c in range(CTX):
        chunk = e[:, c * EMBED_DIM:(c + 1) * EMBED_DIM]
        n2 = jnp.sum(chunk * chunk, axis=1, keepdims=True)
        scale = jnp.minimum(1.0, 1.0 / (jnp.sqrt(n2) + 1e-7))
        acc = acc + chunk * scale
    h = acc * (1.0 / CTX)

    slot = lax.rem(i, NBUF)

    def _out_copy(s, step):
        return pltpu.make_async_copy(
            buf.at[s, :, pl.ds(0, V_MAIN)],
            out_hbm.at[pl.ds(step * B_BLK, B_BLK), pl.ds(0, V_MAIN)],
            sems.at[s])

    @pl.when(i >= NBUF)
    def _reclaim():
        _out_copy(slot, i - NBUF).wait()

    res = lax.dot_general(
        h, wv[...],
        dimension_numbers=(((1,), (0,)), ((), ())),
        preferred_element_type=jnp.float32,
    ) + b_ref[...]
    buf[slot] = res
    rem[pl.ds(i * B_BLK, B_BLK)] = res[:, V_MAIN:]
    _out_copy(slot, i).start()

    @pl.when(i == NSTEP - 1)
    def _drain():
        for k in range(NBUF):
            step = NSTEP - NBUF + ((k - NSTEP) % NBUF)
            _out_copy(k, step).wait()
        cp = pltpu.make_async_copy(
            rem, out_hbm.at[:, pl.ds(V_MAIN, V_REM)], wsem)
        cp.start()
        cp.wait()


def kernel(x, table, W, b):
    idx = x.reshape(TOTAL_IDX)
    emb = _gather_call()(idx, table)
    emb2d = emb.reshape(BATCH, CTX * EMBED_DIM)

    logits = pl.pallas_call(
        _tc_body,
        grid=(NSTEP,),
        in_specs=[
            pl.BlockSpec((B_BLK, CTX * EMBED_DIM), lambda i: (i, 0)),
            pl.BlockSpec(memory_space=pl.ANY),
            pl.BlockSpec((1, VOCAB), lambda i: (0, 0)),
        ],
        out_specs=pl.BlockSpec(memory_space=pl.ANY),
        out_shape=jax.ShapeDtypeStruct((BATCH, VOCAB), jnp.float32),
        scratch_shapes=[
            pltpu.VMEM((EMBED_DIM, VOCAB), jnp.float32),
            pltpu.VMEM((NBUF, B_BLK, VOCAB), jnp.float32),
            pltpu.VMEM((BATCH, V_REM), jnp.float32),
            pltpu.SemaphoreType.DMA,
            pltpu.SemaphoreType.DMA((NBUF,)),
        ],
        compiler_params=pltpu.CompilerParams(
            dimension_semantics=("arbitrary",)),
    )(emb2d, _transpose_w(W), b.reshape(1, VOCAB))
    return logits


# speedup vs baseline: 1.0500x; 1.0012x over previous
import functools

import jax
import jax.numpy as jnp
from jax import lax
from jax.experimental import pallas as pl
from jax.experimental.pallas import tpu as pltpu
from jax.experimental.pallas import tpu_sc as plsc

VOCAB = 100000
EMBED_DIM = 32
BATCH = 1024
CTX = 4

NUM_SC = 2
NUM_SUBCORES = 16
NUM_WORKERS = NUM_SC * NUM_SUBCORES
TOTAL_IDX = BATCH * CTX
IDX_PER_W = TOTAL_IDX // NUM_WORKERS

V_BLK = 2048
W_CHUNK = 8192


def _sc_gather(idx_hbm, table_hbm, out_hbm, idx_v, rows_v, sem):
    wid = lax.axis_index("s") * NUM_SC + lax.axis_index("c")
    base = wid * IDX_PER_W
    pltpu.sync_copy(idx_hbm.at[pl.ds(base, IDX_PER_W)], idx_v)
    pltpu.async_copy(table_hbm.at[idx_v], rows_v, sem).wait()
    pltpu.sync_copy(rows_v, out_hbm.at[pl.ds(base, IDX_PER_W)])


@functools.cache
def _gather_call():
    return pl.kernel(
        _sc_gather,
        out_type=jax.ShapeDtypeStruct((TOTAL_IDX, EMBED_DIM), jnp.float32),
        mesh=plsc.VectorSubcoreMesh(core_axis_name="c", subcore_axis_name="s"),
        scratch_types=[
            pltpu.VMEM((IDX_PER_W,), jnp.int32),
            pltpu.VMEM((IDX_PER_W, EMBED_DIM), jnp.float32),
            pltpu.SemaphoreType.DMA,
        ],
        compiler_params=pltpu.CompilerParams(use_tc_tiling_on_sc=False),
    )


def _tr_body(w_ref, wt_ref):
    wt_ref[...] = w_ref[...].T


def _transpose_w(W):
    grid = (VOCAB + W_CHUNK - 1) // W_CHUNK
    return pl.pallas_call(
        _tr_body,
        grid=(grid,),
        in_specs=[pl.BlockSpec((W_CHUNK, EMBED_DIM), lambda i: (i, 0))],
        out_specs=pl.BlockSpec((EMBED_DIM, W_CHUNK), lambda i: (0, i)),
        out_shape=jax.ShapeDtypeStruct((EMBED_DIM, VOCAB), jnp.float32),
    )(W)


def _mm_body(emb_ref, wt_ref, b_ref, out_ref, h_ref):
    @pl.when(pl.program_id(0) == 0)
    def _compute_h():
        e = emb_ref[...]
        acc = jnp.zeros((BATCH, EMBED_DIM), jnp.float32)
        for c in range(CTX):
            chunk = e[:, c * EMBED_DIM:(c + 1) * EMBED_DIM]
            n2 = jnp.sum(chunk * chunk, axis=1, keepdims=True)
            scale = jnp.minimum(1.0, 1.0 / (jnp.sqrt(n2) + 1e-7))
            acc = acc + chunk * scale
        h_ref[...] = acc * (1.0 / CTX)

    out_ref[...] = lax.dot_general(
        h_ref[...], wt_ref[...],
        dimension_numbers=(((1,), (0,)), ((), ())),
        preferred_element_type=jnp.float32,
    ) + b_ref[...]


def kernel(x, table, W, b):
    idx = x.reshape(TOTAL_IDX)
    emb = _gather_call()(idx, table)
    emb2d = emb.reshape(BATCH, CTX * EMBED_DIM)
    wt = _transpose_w(W)

    grid = (VOCAB + V_BLK - 1) // V_BLK
    logits = pl.pallas_call(
        _mm_body,
        grid=(grid,),
        in_specs=[
            pl.BlockSpec((BATCH, CTX * EMBED_DIM), lambda j: (0, 0)),
            pl.BlockSpec((EMBED_DIM, V_BLK), lambda j: (0, j)),
            pl.BlockSpec((1, V_BLK), lambda j: (0, j)),
        ],
        out_specs=pl.BlockSpec((BATCH, V_BLK), lambda j: (0, j)),
        out_shape=jax.ShapeDtypeStruct((BATCH, VOCAB), jnp.float32),
        scratch_shapes=[pltpu.VMEM((BATCH, EMBED_DIM), jnp.float32)],
    )(emb2d, wt, b.reshape(1, VOCAB))
    return logits


# speedup vs baseline: 1.0504x; 1.0004x over previous
import functools

import jax
import jax.numpy as jnp
from jax import lax
from jax.experimental import pallas as pl
from jax.experimental.pallas import tpu as pltpu
from jax.experimental.pallas import tpu_sc as plsc

VOCAB = 100000
EMBED_DIM = 32
BATCH = 1024
CTX = 4

NUM_SC = 2
NUM_SUBCORES = 16
NUM_WORKERS = NUM_SC * NUM_SUBCORES
TOTAL_IDX = BATCH * CTX
IDX_PER_W = TOTAL_IDX // NUM_WORKERS

V_BLK = 4096
W_CHUNK = 8192


def _sc_gather(idx_hbm, table_hbm, out_hbm, idx_v, rows_v, sem):
    wid = lax.axis_index("s") * NUM_SC + lax.axis_index("c")
    base = wid * IDX_PER_W
    pltpu.sync_copy(idx_hbm.at[pl.ds(base, IDX_PER_W)], idx_v)
    pltpu.async_copy(table_hbm.at[idx_v], rows_v, sem).wait()
    pltpu.sync_copy(rows_v, out_hbm.at[pl.ds(base, IDX_PER_W)])


@functools.cache
def _gather_call():
    return pl.kernel(
        _sc_gather,
        out_type=jax.ShapeDtypeStruct((TOTAL_IDX, EMBED_DIM), jnp.float32),
        mesh=plsc.VectorSubcoreMesh(core_axis_name="c", subcore_axis_name="s"),
        scratch_types=[
            pltpu.VMEM((IDX_PER_W,), jnp.int32),
            pltpu.VMEM((IDX_PER_W, EMBED_DIM), jnp.float32),
            pltpu.SemaphoreType.DMA,
        ],
        compiler_params=pltpu.CompilerParams(use_tc_tiling_on_sc=False),
    )


def _tr_body(w_ref, wt_ref):
    wt_ref[...] = w_ref[...].T


def _transpose_w(W):
    grid = (VOCAB + W_CHUNK - 1) // W_CHUNK
    return pl.pallas_call(
        _tr_body,
        grid=(grid,),
        in_specs=[pl.BlockSpec((W_CHUNK, EMBED_DIM), lambda i: (i, 0))],
        out_specs=pl.BlockSpec((EMBED_DIM, W_CHUNK), lambda i: (0, i)),
        out_shape=jax.ShapeDtypeStruct((EMBED_DIM, VOCAB), jnp.float32),
    )(W)


def _mm_body(emb_ref, wt_ref, b_ref, out_ref, h_ref):
    @pl.when(pl.program_id(0) == 0)
    def _compute_h():
        e = emb_ref[...]
        acc = jnp.zeros((BATCH, EMBED_DIM), jnp.float32)
        for c in range(CTX):
            chunk = e[:, c * EMBED_DIM:(c + 1) * EMBED_DIM]
            n2 = jnp.sum(chunk * chunk, axis=1, keepdims=True)
            scale = jnp.minimum(1.0, 1.0 / (jnp.sqrt(n2) + 1e-7))
            acc = acc + chunk * scale
        h_ref[...] = acc * (1.0 / CTX)

    out_ref[...] = lax.dot_general(
        h_ref[...], wt_ref[...],
        dimension_numbers=(((1,), (0,)), ((), ())),
        preferred_element_type=jnp.float32,
    ) + b_ref[...]


def kernel(x, table, W, b):
    idx = x.reshape(TOTAL_IDX)
    emb = _gather_call()(idx, table)
    emb2d = emb.reshape(BATCH, CTX * EMBED_DIM)
    wt = _transpose_w(W)

    grid = (VOCAB + V_BLK - 1) // V_BLK
    logits = pl.pallas_call(
        _mm_body,
        grid=(grid,),
        in_specs=[
            pl.BlockSpec((BATCH, CTX * EMBED_DIM), lambda j: (0, 0)),
            pl.BlockSpec((EMBED_DIM, V_BLK), lambda j: (0, j)),
            pl.BlockSpec((1, V_BLK), lambda j: (0, j)),
        ],
        out_specs=pl.BlockSpec((BATCH, V_BLK), lambda j: (0, j)),
        out_shape=jax.ShapeDtypeStruct((BATCH, VOCAB), jnp.float32),
        scratch_shapes=[pltpu.VMEM((BATCH, EMBED_DIM), jnp.float32)],
    )(emb2d, wt, b.reshape(1, VOCAB))
    return logits


# speedup vs baseline: 1.1573x; 1.1018x over previous
import functools

import jax
import jax.numpy as jnp
from jax import lax
from jax.experimental import pallas as pl
from jax.experimental.pallas import tpu as pltpu
from jax.experimental.pallas import tpu_sc as plsc

VOCAB = 100000
EMBED_DIM = 32
BATCH = 1024
CTX = 4

NUM_SC = 2
NUM_SUBCORES = 16
NUM_WORKERS = NUM_SC * NUM_SUBCORES
TOTAL_IDX = BATCH * CTX
IDX_PER_W = TOTAL_IDX // NUM_WORKERS

V_BLK = 1024


def _sc_gather(idx_hbm, table_hbm, out_hbm, idx_v, rows_v, sem):
    wid = lax.axis_index("s") * NUM_SC + lax.axis_index("c")
    base = wid * IDX_PER_W
    pltpu.sync_copy(idx_hbm.at[pl.ds(base, IDX_PER_W)], idx_v)
    pltpu.async_copy(table_hbm.at[idx_v], rows_v, sem).wait()
    pltpu.sync_copy(rows_v, out_hbm.at[pl.ds(base, IDX_PER_W)])


@functools.cache
def _gather_call():
    return pl.kernel(
        _sc_gather,
        out_type=jax.ShapeDtypeStruct((TOTAL_IDX, EMBED_DIM), jnp.float32),
        mesh=plsc.VectorSubcoreMesh(core_axis_name="c", subcore_axis_name="s"),
        scratch_types=[
            pltpu.VMEM((IDX_PER_W,), jnp.int32),
            pltpu.VMEM((IDX_PER_W, EMBED_DIM), jnp.float32),
            pltpu.SemaphoreType.DMA,
        ],
        compiler_params=pltpu.CompilerParams(use_tc_tiling_on_sc=False),
    )


B_BLK = 32


def _tc_body(emb_ref, w_ref, b_ref, out_ref):
    e = emb_ref[...]
    acc = jnp.zeros((B_BLK, EMBED_DIM), jnp.float32)
    for c in range(CTX):
        chunk = e[:, c * EMBED_DIM:(c + 1) * EMBED_DIM]
        n2 = jnp.sum(chunk * chunk, axis=1, keepdims=True)
        scale = jnp.minimum(1.0, 1.0 / (jnp.sqrt(n2) + 1e-7))
        acc = acc + chunk * scale
    h = acc * (1.0 / CTX)
    out_ref[...] = lax.dot_general(
        h, w_ref[...],
        dimension_numbers=(((1,), (0,)), ((), ())),
        preferred_element_type=jnp.float32,
    ) + b_ref[...]


def kernel(x, table, W, b):
    idx = x.reshape(TOTAL_IDX)
    emb = _gather_call()(idx, table)
    emb2d = emb.reshape(BATCH, CTX * EMBED_DIM)

    grid = BATCH // B_BLK
    logits = pl.pallas_call(
        _tc_body,
        grid=(grid,),
        in_specs=[
            pl.BlockSpec((B_BLK, CTX * EMBED_DIM), lambda i: (i, 0)),
            pl.BlockSpec((EMBED_DIM, VOCAB), lambda i: (0, 0)),
            pl.BlockSpec((1, VOCAB), lambda i: (0, 0)),
        ],
        out_specs=pl.BlockSpec((B_BLK, VOCAB), lambda i: (i, 0)),
        out_shape=jax.ShapeDtypeStruct((BATCH, VOCAB), jnp.float32),
    )(emb2d, W.T, b.reshape(1, VOCAB))
    return logits
